# baseline (speedup 1.0000x reference)
import jax
import jax.numpy as jnp
from jax import lax
from jax.experimental import pallas as pl

B, C, H, W = 64, 384, 32, 32
HW = H * W
E = 8
RED = 48
SUB = HW // 128


def _body(x_ref, w1e_ref, b1_ref, w2t_ref, b2t_ref, out_ref):
    xs = x_ref[0]
    m = jnp.dot(w1e_ref[...], xs, preferred_element_type=jnp.float32)
    h = jnp.sum(m, axis=1, keepdims=True) * (1.0 / HW) + b1_ref[...]
    h = h * jax.nn.sigmoid(h)
    logits = jnp.dot(w2t_ref[...], h, preferred_element_type=jnp.float32)
    logits = logits + b2t_ref[...]
    logits = logits - jnp.max(logits, axis=0, keepdims=True)
    ex = jnp.exp(logits)
    w = ex / jnp.sum(ex, axis=0, keepdims=True)
    idx = lax.broadcasted_iota(jnp.int32, (E, 1), 0)
    m1 = jnp.max(w, axis=0, keepdims=True)
    i1 = jnp.min(jnp.where(w == m1, idx, E), axis=0, keepdims=True)
    w_rest = jnp.where(idx == i1, -jnp.inf, w)
    m2 = jnp.max(w_rest, axis=0, keepdims=True)
    i2 = jnp.min(jnp.where(w_rest == m2, idx, E), axis=0, keepdims=True)
    mask = (idx == i1) | (idx == i2)
    wsel = jnp.where(mask, w, 0.0)
    wn = wsel / (jnp.sum(wsel, axis=0, keepdims=True) + 1e-8)
    out_ref[...] = jnp.broadcast_to(wn[:, :, None], (E, E, 128)).reshape(E * E, 128)


@jax.jit
def kernel(x, W1, b1, W2, b2):
    xv = x.reshape(B, C * SUB, 128)
    w1e = jnp.repeat(W1, SUB, axis=0).T
    out = pl.pallas_call(
        _body,
        grid=(B,),
        in_specs=[
            pl.BlockSpec((1, C * SUB, 128), lambda i: (i, 0, 0)),
            pl.BlockSpec((RED, C * SUB), lambda i: (0, 0)),
            pl.BlockSpec((RED, 1), lambda i: (0, 0)),
            pl.BlockSpec((E, RED), lambda i: (0, 0)),
            pl.BlockSpec((E, 1), lambda i: (0, 0)),
        ],
        out_specs=pl.BlockSpec((E * SUB, 128), lambda i: (i, 0)),
        out_shape=jax.ShapeDtypeStruct((B * E * SUB, 128), jnp.float32),
    )(xv, w1e, b1.reshape(RED, 1), W2.T, b2.reshape(E, 1))
    return out.reshape(B, E, H, W)
